# baseline (speedup 1.0000x reference)
import functools

import jax
import jax.numpy as jnp
from jax import lax
from jax.experimental import pallas as pl
from jax.experimental.pallas import tpu as pltpu
from jax.experimental.pallas import tpu_sc as plsc

B, S, D = 4, 4096, 1024
NC, NS = 2, 16
NW = NC * NS
ROWS_PER_W = S // NW
CH = 16
NCHUNK = ROWS_PER_W // CH
CHE = CH * D
NU = NCHUNK * B
NB = 5
NE = 2
UNROLL = 8


def _add_unit(buf, emb):
    @plsc.parallel_loop(0, CHE, step=16, unroll=UNROLL)
    def _(i):
        plsc.addupdate(buf.at[pl.ds(i, 16)], emb[pl.ds(i, 16)])


def _pos_add_sc(in_hbm, emb_hbm, out_hbm, *scratch):
    embs = scratch[:NE]
    bufs = scratch[NE:NE + NB]
    sems_e = scratch[NE + NB:NE + NB + NE]
    sems_i = scratch[NE + NB + NE:NE + NB + NE + NB]
    sems_o = scratch[NE + NB + NE + NB:]

    wid = lax.axis_index("s") * NC + lax.axis_index("c")
    base = wid * (ROWS_PER_W * D)

    def start_emb(c):
        return pltpu.async_copy(
            emb_hbm.at[pl.ds(base + c * CHE, CHE)], embs[c % NE], sems_e[c % NE])

    def start_in(u):
        c, b = divmod(u, B)
        x0 = b * (S * D) + base + c * CHE
        return pltpu.async_copy(in_hbm.at[pl.ds(x0, CHE)], bufs[u % NB],
                                sems_i[u % NB])

    def start_out(u):
        c, b = divmod(u, B)
        x0 = b * (S * D) + base + c * CHE
        return pltpu.async_copy(bufs[u % NB], out_hbm.at[pl.ds(x0, CHE)],
                                sems_o[u % NB])

    e_desc = {c: start_emb(c) for c in range(min(NE, NCHUNK))}
    i_desc = {u: start_in(u) for u in range(min(NB - 1, NU))}
    o_desc = {}
    o_waited = set()

    for u in range(NU):
        c, b = divmod(u, B)
        if b == 0:
            e_desc[c].wait()
        i_desc[u].wait()
        _add_unit(bufs[u % NB], embs[c % NE])
        o_desc[u] = start_out(u)
        v = u + NB - 1
        if v < NU:
            if u >= 1:
                o_desc[u - 1].wait()
                o_waited.add(u - 1)
            i_desc[v] = start_in(v)
        if b == B - 1 and c + NE < NCHUNK:
            e_desc[c + NE] = start_emb(c + NE)

    for u in range(NU):
        if u not in o_waited:
            o_desc[u].wait()


@jax.jit
def _pos_add(flat_inputs, flat_emb):
    return pl.kernel(
        _pos_add_sc,
        out_type=jax.ShapeDtypeStruct((B * S * D,), jnp.float32),
        mesh=plsc.VectorSubcoreMesh(core_axis_name="c", subcore_axis_name="s"),
        scratch_types=(
            [pltpu.VMEM((CHE,), jnp.float32) for _ in range(NE)]
            + [pltpu.VMEM((CHE,), jnp.float32) for _ in range(NB)]
            + [pltpu.SemaphoreType.DMA for _ in range(NE + NB + NB)]
        ),
    )(flat_inputs, flat_emb)


def kernel(inputs, embedding):
    b, s, d = inputs.shape
    out = _pos_add(inputs.reshape(-1), embedding[:s].reshape(-1))
    return out.reshape(b, s, d)


# speedup vs baseline: 2.8094x; 2.8094x over previous
import functools

import jax
import jax.numpy as jnp
from jax import lax
from jax.experimental import pallas as pl
from jax.experimental.pallas import tpu as pltpu
from jax.experimental.pallas import tpu_sc as plsc

B, S, D = 4, 4096, 1024
NC, NS = 2, 16
NW = NC * NS
ROWS_PER_W = S // NW
CH = 16
NCHUNK = ROWS_PER_W // CH
NU = NCHUNK * B
NB = 5
NE = 2
UNROLL = 8
LANES_PER_ROW = D // 16


def _add_unit(buf, emb):
    @plsc.parallel_loop(0, CH * LANES_PER_ROW, step=1, unroll=UNROLL)
    def _(i):
        r = i // LANES_PER_ROW
        o = (i % LANES_PER_ROW) * 16
        plsc.addupdate(buf.at[r, pl.ds(o, 16)], emb[r, pl.ds(o, 16)])


def _pos_add_sc(in_hbm, emb_hbm, out_hbm, *scratch):
    embs = scratch[:NE]
    bufs = scratch[NE:NE + NB]
    sems_e = scratch[NE + NB:NE + NB + NE]
    sems_i = scratch[NE + NB + NE:NE + NB + NE + NB]
    sems_o = scratch[NE + NB + NE + NB:]

    wid = lax.axis_index("s") * NC + lax.axis_index("c")
    base = wid * ROWS_PER_W

    def start_emb(c):
        return pltpu.async_copy(
            emb_hbm.at[pl.ds(base + c * CH, CH)], embs[c % NE], sems_e[c % NE])

    def start_in(u):
        c, b = divmod(u, B)
        r0 = b * S + base + c * CH
        return pltpu.async_copy(in_hbm.at[pl.ds(r0, CH)], bufs[u % NB],
                                sems_i[u % NB])

    def start_out(u):
        c, b = divmod(u, B)
        r0 = b * S + base + c * CH
        return pltpu.async_copy(bufs[u % NB], out_hbm.at[pl.ds(r0, CH)],
                                sems_o[u % NB])

    e_desc = {c: start_emb(c) for c in range(min(NE, NCHUNK))}
    i_desc = {u: start_in(u) for u in range(min(NB - 1, NU))}
    o_desc = {}
    o_waited = set()

    for u in range(NU):
        c, b = divmod(u, B)
        if b == 0:
            e_desc[c].wait()
        i_desc[u].wait()
        _add_unit(bufs[u % NB], embs[c % NE])
        o_desc[u] = start_out(u)
        v = u + NB - 1
        if v < NU:
            if u >= 1:
                o_desc[u - 1].wait()
                o_waited.add(u - 1)
            i_desc[v] = start_in(v)
        if b == B - 1 and c + NE < NCHUNK:
            e_desc[c + NE] = start_emb(c + NE)

    for u in range(NU):
        if u not in o_waited:
            o_desc[u].wait()


@jax.jit
def _pos_add(inputs2d, emb):
    return pl.kernel(
        _pos_add_sc,
        out_type=jax.ShapeDtypeStruct((B * S, D), jnp.float32),
        mesh=plsc.VectorSubcoreMesh(core_axis_name="c", subcore_axis_name="s"),
        scratch_types=(
            [pltpu.VMEM((CH, D), jnp.float32) for _ in range(NE)]
            + [pltpu.VMEM((CH, D), jnp.float32) for _ in range(NB)]
            + [pltpu.SemaphoreType.DMA for _ in range(NE + NB + NB)]
        ),
    )(inputs2d, emb)


def kernel(inputs, embedding):
    b, s, d = inputs.shape
    out = _pos_add(inputs.reshape(b * s, d), embedding[:s])
    return out.reshape(b, s, d)
